# baseline (speedup 1.0000x reference)
import functools

import jax
import jax.numpy as jnp
from jax import lax
from jax.experimental import pallas as pl
from jax.experimental.pallas import tpu as pltpu
from jax.experimental.pallas import tpu_sc as plsc

EMBED = 512
HIDDEN = 1024

_NUM_WORKERS = 32
_CHUNK = 128


def _sc_gather(table, ids3d):
    nw, n_ch, ch = ids3d.shape
    b_total = nw * n_ch * ch
    b_per_w = n_ch * ch
    mesh = plsc.VectorSubcoreMesh(core_axis_name="c", subcore_axis_name="s")

    @functools.partial(
        pl.kernel,
        out_type=jax.ShapeDtypeStruct((b_total, EMBED), jnp.float32),
        mesh=mesh,
        scratch_types=[
            pltpu.VMEM((n_ch, ch), jnp.int32),
            pltpu.VMEM((ch, EMBED), jnp.float32),
            pltpu.SemaphoreType.DMA,
        ],
    )
    def k(table_hbm, idx_hbm, out_hbm, idx_v, rows_v, sem):
        wid = lax.axis_index("s") * 2 + lax.axis_index("c")
        base = wid * b_per_w
        pltpu.sync_copy(idx_hbm.at[wid], idx_v)
        for c in range(n_ch):
            pltpu.async_copy(table_hbm.at[idx_v.at[c]], rows_v, sem).wait()
            pltpu.sync_copy(rows_v, out_hbm.at[pl.ds(base + c * ch, ch)])

    return k(table, ids3d)


def _tc_project(x, pos_row, w, b):
    n = x.shape[0]
    tb = 256

    def body(x_ref, pos_ref, w_ref, b_ref, o_ref):
        xx = (x_ref[...] + pos_ref[...]).astype(jnp.bfloat16)
        o_ref[...] = (
            jnp.dot(xx, w_ref[...], preferred_element_type=jnp.float32) + b_ref[...]
        )

    return pl.pallas_call(
        body,
        grid=(n // tb,),
        in_specs=[
            pl.BlockSpec((tb, EMBED), lambda i: (i, 0)),
            pl.BlockSpec((1, EMBED), lambda i: (0, 0)),
            pl.BlockSpec((EMBED, HIDDEN), lambda i: (0, 0)),
            pl.BlockSpec((1, HIDDEN), lambda i: (0, 0)),
        ],
        out_specs=pl.BlockSpec((tb, HIDDEN), lambda i: (i, 0)),
        out_shape=jax.ShapeDtypeStruct((n, HIDDEN), jnp.float32),
    )(x, pos_row, w, b)


def kernel(input_ids, embed_tokens_w, embed_positions_w, proj_w, proj_b):
    batch, seq = input_ids.shape
    b_total = batch * seq
    b_per_w = b_total // _NUM_WORKERS
    n_ch = b_per_w // _CHUNK
    ids3d = input_ids.reshape(_NUM_WORKERS, n_ch, _CHUNK).astype(jnp.int32)

    gathered = _sc_gather(embed_tokens_w, ids3d)

    pos_row = lax.dynamic_slice_in_dim(embed_positions_w, 1, 1, axis=0)
    out = _tc_project(
        gathered, pos_row, proj_w.astype(jnp.bfloat16), proj_b.reshape(1, HIDDEN)
    )
    return out.reshape(batch, seq, HIDDEN)


# speedup vs baseline: 1.2395x; 1.2395x over previous
import functools

import jax
import jax.numpy as jnp
from jax import lax
from jax.experimental import pallas as pl
from jax.experimental.pallas import tpu as pltpu
from jax.experimental.pallas import tpu_sc as plsc

EMBED = 512
HIDDEN = 1024

_NUM_WORKERS = 32
_CHUNK = 128


def _sc_gather(table, ids3d):
    nw, n_ch, ch = ids3d.shape
    b_total = nw * n_ch * ch
    b_per_w = n_ch * ch
    mesh = plsc.VectorSubcoreMesh(core_axis_name="c", subcore_axis_name="s")

    @functools.partial(
        pl.kernel,
        out_type=jax.ShapeDtypeStruct((b_total, EMBED), jnp.float32),
        mesh=mesh,
        scratch_types=[
            pltpu.VMEM((n_ch, ch), jnp.int32),
            pltpu.VMEM((ch, EMBED), jnp.float32),
            pltpu.SemaphoreType.DMA,
        ],
    )
    def k(table_hbm, idx_hbm, out_hbm, idx_v, rows_v, sem):
        wid = lax.axis_index("s") * 2 + lax.axis_index("c")
        base = wid * b_per_w
        pltpu.sync_copy(idx_hbm.at[wid], idx_v)
        for c in range(n_ch):
            pltpu.async_copy(table_hbm.at[idx_v.at[c]], rows_v, sem).wait()
            pltpu.sync_copy(rows_v, out_hbm.at[pl.ds(base + c * ch, ch)])

    return k(table, ids3d)


def _tc_project(x, pos_row, w, b):
    n = x.shape[0]
    tb = 1024

    def body(x_ref, pos_ref, w_ref, b_ref, o_ref):
        xx = (x_ref[...] + pos_ref[...]).astype(jnp.bfloat16)
        o_ref[...] = (
            jnp.dot(xx, w_ref[...], preferred_element_type=jnp.float32) + b_ref[...]
        )

    return pl.pallas_call(
        body,
        grid=(n // tb,),
        in_specs=[
            pl.BlockSpec((tb, EMBED), lambda i: (i, 0)),
            pl.BlockSpec((1, EMBED), lambda i: (0, 0)),
            pl.BlockSpec((EMBED, HIDDEN), lambda i: (0, 0)),
            pl.BlockSpec((1, HIDDEN), lambda i: (0, 0)),
        ],
        out_specs=pl.BlockSpec((tb, HIDDEN), lambda i: (i, 0)),
        out_shape=jax.ShapeDtypeStruct((n, HIDDEN), jnp.float32),
    )(x, pos_row, w, b)


def kernel(input_ids, embed_tokens_w, embed_positions_w, proj_w, proj_b):
    batch, seq = input_ids.shape
    b_total = batch * seq
    b_per_w = b_total // _NUM_WORKERS
    n_ch = b_per_w // _CHUNK
    ids3d = input_ids.reshape(_NUM_WORKERS, n_ch, _CHUNK).astype(jnp.int32)

    gathered = _sc_gather(embed_tokens_w, ids3d)

    pos_row = lax.dynamic_slice_in_dim(embed_positions_w, 1, 1, axis=0)
    out = _tc_project(
        gathered, pos_row, proj_w.astype(jnp.bfloat16), proj_b.reshape(1, HIDDEN)
    )
    return out.reshape(batch, seq, HIDDEN)


# speedup vs baseline: 1.2729x; 1.0270x over previous
import functools

import jax
import jax.numpy as jnp
from jax import lax
from jax.experimental import pallas as pl
from jax.experimental.pallas import tpu as pltpu
from jax.experimental.pallas import tpu_sc as plsc

EMBED = 512
HIDDEN = 1024

_NUM_WORKERS = 32
_CHUNK = 128


def _sc_gather(table, ids3d):
    nw, n_ch, ch = ids3d.shape
    b_total = nw * n_ch * ch
    b_per_w = n_ch * ch
    mesh = plsc.VectorSubcoreMesh(core_axis_name="c", subcore_axis_name="s")

    @functools.partial(
        pl.kernel,
        out_type=jax.ShapeDtypeStruct((b_total, EMBED), jnp.float32),
        mesh=mesh,
        scratch_types=[
            pltpu.VMEM((n_ch, ch), jnp.int32),
            pltpu.VMEM((ch, EMBED), jnp.float32),
            pltpu.SemaphoreType.DMA,
        ],
    )
    def k(table_hbm, idx_hbm, out_hbm, idx_v, rows_v, sem):
        wid = lax.axis_index("s") * 2 + lax.axis_index("c")
        base = wid * b_per_w
        pltpu.sync_copy(idx_hbm.at[wid], idx_v)
        for c in range(n_ch):
            pltpu.async_copy(table_hbm.at[idx_v.at[c]], rows_v, sem).wait()
            pltpu.sync_copy(rows_v, out_hbm.at[pl.ds(base + c * ch, ch)])

    return k(table, ids3d)


def _tc_project(x, pos_row, w, b):
    n = x.shape[0]
    tb = 2048

    def body(x_ref, pos_ref, w_ref, b_ref, o_ref):
        xx = (x_ref[...] + pos_ref[...]).astype(jnp.bfloat16)
        o_ref[...] = (
            jnp.dot(xx, w_ref[...], preferred_element_type=jnp.float32) + b_ref[...]
        )

    return pl.pallas_call(
        body,
        grid=(n // tb,),
        in_specs=[
            pl.BlockSpec((tb, EMBED), lambda i: (i, 0)),
            pl.BlockSpec((1, EMBED), lambda i: (0, 0)),
            pl.BlockSpec((EMBED, HIDDEN), lambda i: (0, 0)),
            pl.BlockSpec((1, HIDDEN), lambda i: (0, 0)),
        ],
        out_specs=pl.BlockSpec((tb, HIDDEN), lambda i: (i, 0)),
        out_shape=jax.ShapeDtypeStruct((n, HIDDEN), jnp.float32),
    )(x, pos_row, w, b)


def kernel(input_ids, embed_tokens_w, embed_positions_w, proj_w, proj_b):
    batch, seq = input_ids.shape
    b_total = batch * seq
    b_per_w = b_total // _NUM_WORKERS
    n_ch = b_per_w // _CHUNK
    ids3d = input_ids.reshape(_NUM_WORKERS, n_ch, _CHUNK).astype(jnp.int32)

    gathered = _sc_gather(embed_tokens_w, ids3d)

    pos_row = lax.dynamic_slice_in_dim(embed_positions_w, 1, 1, axis=0)
    out = _tc_project(
        gathered, pos_row, proj_w.astype(jnp.bfloat16), proj_b.reshape(1, HIDDEN)
    )
    return out.reshape(batch, seq, HIDDEN)
